# baseline (speedup 1.0000x reference)
import functools

import jax
import jax.numpy as jnp
from jax import lax
from jax.experimental import pallas as pl
from jax.experimental.pallas import tpu as pltpu
from jax.experimental.pallas import tpu_sc as plsc

D_MODEL = 128
N_ROWS = 256

NC = 2
NS = 16
L = 16
NW = NC * NS

_B = 4096 * 50
_PER_W = _B // NW
_CH = 128
_RING = 5
_NSTEP = _PER_W // (_RING * _CH)


def _sc_lookup(px, py, tabm):
    mesh = plsc.VectorSubcoreMesh(core_axis_name="c", subcore_axis_name="s")

    @functools.partial(
        pl.kernel,
        mesh=mesh,
        out_type=jax.ShapeDtypeStruct((_B, D_MODEL), jnp.float32),
        scratch_types=(
            [pltpu.VMEM_SHARED((2 * N_ROWS, D_MODEL), jnp.float32),
             pltpu.VMEM((_PER_W,), jnp.int32),
             pltpu.VMEM((_PER_W,), jnp.int32)]
            + [pltpu.VMEM((_CH, D_MODEL), jnp.float32)] * _RING
            + [pltpu.SemaphoreType.DMA] * (2 * _RING)
        ),
        compiler_params=pltpu.CompilerParams(needs_layout_passes=False),
    )
    def k(px_hbm, py_hbm, tab_hbm, out_hbm, tabS, pxv, pyv, *bufs_sems):
        bufs = bufs_sems[:_RING]
        gsem = bufs_sems[_RING:2 * _RING]
        wsem = bufs_sems[2 * _RING:]
        sid = lax.axis_index("s")
        wid = sid * NC + lax.axis_index("c")
        base = wid * _PER_W

        @pl.when(sid == 0)
        def _stage():
            pltpu.sync_copy(tab_hbm, tabS)

        pltpu.sync_copy(px_hbm.at[pl.ds(base, _PER_W)], pxv)
        pltpu.sync_copy(py_hbm.at[pl.ds(base, _PER_W)], pyv)

        def addy(i, carry):
            pyv[pl.ds(i * L, L)] = pyv[pl.ds(i * L, L)] + N_ROWS
            return carry

        lax.fori_loop(0, _PER_W // L, addy, 0)
        plsc.subcore_barrier()

        def step(t, carry):
            o0 = t * (_RING * _CH)
            gx = []
            for i in range(_RING):
                @pl.when(t > 0)
                def _w(i=i):
                    pltpu.make_async_copy(
                        bufs[i], out_hbm.at[pl.ds(base, _CH)],
                        wsem[i]).wait()
                gx.append(pltpu.async_copy(
                    tab_hbm.at[pxv.at[pl.ds(o0 + i * _CH, _CH)]],
                    bufs[i], gsem[i]))
            gy = []
            for i in range(_RING):
                gx[i].wait()
                gy.append(pltpu.async_copy(
                    tabS.at[pyv.at[pl.ds(o0 + i * _CH, _CH)]],
                    bufs[i], gsem[i], add=True))
            for i in range(_RING):
                gy[i].wait()
                pltpu.async_copy(
                    bufs[i], out_hbm.at[pl.ds(base + o0 + i * _CH, _CH)],
                    wsem[i])
            return carry

        lax.fori_loop(0, _NSTEP, step, 0)
        for i in range(_RING):
            pltpu.make_async_copy(
                bufs[i], out_hbm.at[pl.ds(base, _CH)], wsem[i]).wait()

    return k(px, py, tabm)


def kernel(positions_x, positions_y, pe):
    B, S = positions_x.shape
    px = positions_x.T.reshape(-1).astype(jnp.int32)
    py = positions_y.T.reshape(-1).astype(jnp.int32)
    chan = jnp.arange(D_MODEL) % 4 < 2
    tabm = jnp.concatenate(
        [jnp.where(chan[None, :], pe[:, 0, :], 0.0),
         jnp.where(chan[None, :], 0.0, pe[0, :, :])], axis=0)
    out = _sc_lookup(px, py, tabm)
    return out.reshape(S, B, D_MODEL).transpose(1, 0, 2)


# speedup vs baseline: 2.3056x; 2.3056x over previous
import functools

import jax
import jax.numpy as jnp
from jax import lax
from jax.experimental import pallas as pl
from jax.experimental.pallas import tpu as pltpu
from jax.experimental.pallas import tpu_sc as plsc

D_MODEL = 128
N_ROWS = 256

NC = 2
NS = 16
L = 16
NW = NC * NS

_B = 4096 * 50
_PER_W = _B // NW
_CH = 128
_RING = 5
_NSTEP = _PER_W // (_RING * _CH)


def _sc_lookup(px, py, tabx, taby):
    mesh = plsc.VectorSubcoreMesh(core_axis_name="c", subcore_axis_name="s")

    @functools.partial(
        pl.kernel,
        mesh=mesh,
        out_type=jax.ShapeDtypeStruct((_B, D_MODEL), jnp.float32),
        scratch_types=(
            [pltpu.VMEM_SHARED((N_ROWS, D_MODEL), jnp.float32),
             pltpu.VMEM_SHARED((N_ROWS, D_MODEL), jnp.float32),
             pltpu.VMEM((_PER_W,), jnp.int32),
             pltpu.VMEM((_PER_W,), jnp.int32)]
            + [pltpu.VMEM((_CH, D_MODEL), jnp.float32)] * _RING
            + [pltpu.SemaphoreType.DMA] * (2 * _RING)
        ),
        compiler_params=pltpu.CompilerParams(needs_layout_passes=False),
    )
    def k(px_hbm, py_hbm, tabx_hbm, taby_hbm, out_hbm,
          tabXS, tabYS, pxv, pyv, *bufs_sems):
        bufs = bufs_sems[:_RING]
        gsem = bufs_sems[_RING:2 * _RING]
        wsem = bufs_sems[2 * _RING:]
        sid = lax.axis_index("s")
        wid = sid * NC + lax.axis_index("c")
        base = wid * _PER_W

        @pl.when(sid == 0)
        def _stage_x():
            pltpu.sync_copy(tabx_hbm, tabXS)

        @pl.when(sid == 1)
        def _stage_y():
            pltpu.sync_copy(taby_hbm, tabYS)

        pltpu.sync_copy(px_hbm.at[pl.ds(base, _PER_W)], pxv)
        pltpu.sync_copy(py_hbm.at[pl.ds(base, _PER_W)], pyv)
        plsc.subcore_barrier()

        def step(t, carry):
            o0 = t * (_RING * _CH)
            gx = []
            for i in range(_RING):
                @pl.when(t > 0)
                def _w(i=i):
                    pltpu.make_async_copy(
                        bufs[i], out_hbm.at[pl.ds(base, _CH)],
                        wsem[i]).wait()
                gx.append(pltpu.async_copy(
                    tabXS.at[pxv.at[pl.ds(o0 + i * _CH, _CH)]],
                    bufs[i], gsem[i]))
            gy = []
            for i in range(_RING):
                gx[i].wait()
                gy.append(pltpu.async_copy(
                    tabYS.at[pyv.at[pl.ds(o0 + i * _CH, _CH)]],
                    bufs[i], gsem[i], add=True))
            for i in range(_RING):
                gy[i].wait()
                pltpu.async_copy(
                    bufs[i], out_hbm.at[pl.ds(base + o0 + i * _CH, _CH)],
                    wsem[i])
            return carry

        lax.fori_loop(0, _NSTEP, step, 0)
        for i in range(_RING):
            pltpu.make_async_copy(
                bufs[i], out_hbm.at[pl.ds(base, _CH)], wsem[i]).wait()

    return k(px, py, tabx, taby)


def kernel(positions_x, positions_y, pe):
    B, S = positions_x.shape
    px = positions_x.T.reshape(-1).astype(jnp.int32)
    py = positions_y.T.reshape(-1).astype(jnp.int32)
    chan = jnp.arange(D_MODEL) % 4 < 2
    tabx = jnp.where(chan[None, :], pe[:, 0, :], 0.0)
    taby = jnp.where(chan[None, :], 0.0, pe[0, :, :])
    out = _sc_lookup(px, py, tabx, taby)
    return out.reshape(S, B, D_MODEL).transpose(1, 0, 2)
